# baseline (speedup 1.0000x reference)
import jax
import jax.numpy as jnp
from jax.experimental import pallas as pl


def _gru(m, h, wih, whh, bih, bhh):
    gi = m @ wih + bih
    gh = h @ whh + bhh
    i_r, i_z, i_n = jnp.split(gi, 3, axis=1)
    h_r, h_z, h_n = jnp.split(gh, 3, axis=1)
    r = jax.nn.sigmoid(i_r + h_r)
    zg = jax.nn.sigmoid(i_z + h_z)
    n = jnp.tanh(i_n + r * h_n)
    return (1.0 - zg) * n + zg * h


def _elu(x):
    return jnp.where(x > 0, x, jnp.exp(jnp.minimum(x, 0.0)) - 1.0)


def _mlp_kernel(x_ref, w1, b1, w2, b2, w3, b3, out_ref):
    x = x_ref[...]
    x = _elu(x @ w1[...] + b1[...])
    x = _elu(x @ w2[...] + b2[...])
    out_ref[...] = x @ w3[...] + b3[...]


def kernel(z, edge_index, node_to_subgraph, subgraph_to_graph, node_emb, Wm,
           W_ih, W_hh, b_ih, b_hh, fc1_w, fc1_b, fc2_w, fc2_b, fc3_w, fc3_b):
    L = Wm.shape[0]
    N = z.shape[0]
    S = subgraph_to_graph.shape[0]
    G = 50
    x = jnp.take(node_emb, jnp.zeros_like(z), axis=0)
    src = edge_index[0]
    dst = edge_index[1]
    zf = (z == 1)[:, None]
    for l in range(L):
        m0 = x @ Wm[l, 0]
        m1 = x @ Wm[l, 1]
        m = jnp.where(zf, m1, m0)
        msgs = jnp.take(m, src, axis=0)
        agg = jax.ops.segment_sum(msgs, dst, num_segments=N)
        x = _gru(agg, x, W_ih[l], W_hh[l], b_ih[l], b_hh[l])
    x = jax.ops.segment_sum(x, node_to_subgraph, num_segments=S)
    x = jax.ops.segment_sum(x, subgraph_to_graph, num_segments=G)
    out = pl.pallas_call(
        _mlp_kernel,
        out_shape=jax.ShapeDtypeStruct((G, 1), jnp.float32),
    )(x, fc1_w, fc1_b, fc2_w, fc2_b, fc3_w, fc3_b)
    return out


# speedup vs baseline: 3.4831x; 3.4831x over previous
import functools

import jax
import jax.numpy as jnp
from jax import lax
from jax.experimental import pallas as pl
from jax.experimental.pallas import tpu as pltpu
import jax.experimental.pallas.tpu_sc as plsc

_N = 50000
_E = 800000
_S = 500
_G = 50
_D = 64

_HALF = _N // 2
_TPR = 1664
_RPAD = 16 * _TPR
_NP = 2 * _RPAD
_PAD_OFF = _RPAD - _HALF
_TRASH = _RPAD
_SPROWS = _RPAD + 16
_CHUNK = 128
_EPT = 50048

_PBLK = 64
_PPT = _NP // 32


def _sigmoid(x):
    return 1.0 / (1.0 + jnp.exp(-x))


def _elu(x):
    return jnp.where(x > 0, x, jnp.exp(jnp.minimum(x, 0.0)) - 1.0)


def _idx_body(src_ref, dst_ref, srcp_ref, ld0_ref, ld1_ref):
    src = src_ref[...]
    dst = dst_ref[...]
    srcp_ref[...] = jnp.where(src < _HALF, src, src + _PAD_OFF)
    ld0_ref[...] = jnp.where(dst < _HALF, dst, _TRASH)
    ld1_ref[...] = jnp.where(dst >= _HALF, dst - _HALF, _TRASH)


def _prep_indices(src, dst):
    src2 = src.reshape(6250, 128)
    dst2 = dst.reshape(6250, 128)
    srcp, ld0, ld1 = pl.pallas_call(
        _idx_body,
        out_shape=[jax.ShapeDtypeStruct((6250, 128), jnp.int32)] * 3,
    )(src2, dst2)
    return srcp.reshape(_E), ld0.reshape(_E), ld1.reshape(_E)


_TBLK = 1024
_TGRID = _NP // _TBLK


def _init_body(z_ref, e_ref, w0_ref, w1_ref, x_ref, m_ref):
    e = e_ref[...]
    x_ref[...] = jnp.broadcast_to(e, x_ref.shape)
    m0 = jnp.dot(e, w0_ref[...], preferred_element_type=jnp.float32)
    m1 = jnp.dot(e, w1_ref[...], preferred_element_type=jnp.float32)
    zf = z_ref[...] == 1
    m_ref[...] = jnp.where(zf, m1, m0)


def _init_call(z_p, node_emb, w0, w1):
    row = lambda i: (i, 0)
    fix = lambda i: (0, 0)
    return pl.pallas_call(
        _init_body,
        grid=(_TGRID,),
        in_specs=[
            pl.BlockSpec((_TBLK, 1), row),
            pl.BlockSpec((1, _D), fix),
            pl.BlockSpec((_D, _D), fix),
            pl.BlockSpec((_D, _D), fix),
        ],
        out_specs=[
            pl.BlockSpec((_TBLK, _D), row),
            pl.BlockSpec((_TBLK, _D), row),
        ],
        out_shape=[jax.ShapeDtypeStruct((_NP, _D), jnp.float32)] * 2,
    )(z_p, node_emb, w0, w1)


def _gru_core(a, h, ws):
    wir, wiz, win, whr, whz, whn, bir, biz, bin_, bhr, bhz, bhn = ws
    dot = functools.partial(jnp.dot, preferred_element_type=jnp.float32)
    r = _sigmoid(dot(a, wir) + bir + dot(h, whr) + bhr)
    zg = _sigmoid(dot(a, wiz) + biz + dot(h, whz) + bhz)
    n = jnp.tanh(dot(a, win) + bin_ + r * (dot(h, whn) + bhn))
    return (1.0 - zg) * n + zg * h


def _gru_msg_body(agg_ref, x_ref, z_ref, *refs):
    ws = [r[...] for r in refs[:12]]
    w0, w1, x_out, m_out = refs[12], refs[13], refs[14], refs[15]
    x = _gru_core(agg_ref[...], x_ref[...], ws)
    x_out[...] = x
    dot = functools.partial(jnp.dot, preferred_element_type=jnp.float32)
    m0 = dot(x, w0[...])
    m1 = dot(x, w1[...])
    m_out[...] = jnp.where(z_ref[...] == 1, m1, m0)


def _gru_final_body(agg_ref, x_ref, *refs):
    ws = [r[...] for r in refs[:12]]
    x_out = refs[12]
    x_out[...] = _gru_core(agg_ref[...], x_ref[...], ws)


def _split_gru_weights(W_ih_l, W_hh_l, b_ih_l, b_hh_l):
    out = []
    for w in (W_ih_l, W_hh_l):
        out += [w[:, 0:_D], w[:, _D:2 * _D], w[:, 2 * _D:3 * _D]]
    for b in (b_ih_l, b_hh_l):
        out += [b[0:_D].reshape(1, _D), b[_D:2 * _D].reshape(1, _D),
                b[2 * _D:3 * _D].reshape(1, _D)]
    return out


def _gru_msg_call(agg, x, z_p, ws, w0, w1):
    row = lambda i: (i, 0)
    fix = lambda i: (0, 0)
    wspecs = [pl.BlockSpec((_D, _D), fix)] * 6 + [pl.BlockSpec((1, _D), fix)] * 6
    return pl.pallas_call(
        _gru_msg_body,
        grid=(_TGRID,),
        in_specs=[
            pl.BlockSpec((_TBLK, _D), row),
            pl.BlockSpec((_TBLK, _D), row),
            pl.BlockSpec((_TBLK, 1), row),
            *wspecs,
            pl.BlockSpec((_D, _D), fix),
            pl.BlockSpec((_D, _D), fix),
        ],
        out_specs=[
            pl.BlockSpec((_TBLK, _D), row),
            pl.BlockSpec((_TBLK, _D), row),
        ],
        out_shape=[jax.ShapeDtypeStruct((_NP, _D), jnp.float32)] * 2,
    )(agg, x, z_p, *ws, w0, w1)


def _gru_final_call(agg, x, ws):
    row = lambda i: (i, 0)
    fix = lambda i: (0, 0)
    wspecs = [pl.BlockSpec((_D, _D), fix)] * 6 + [pl.BlockSpec((1, _D), fix)] * 6
    return pl.pallas_call(
        _gru_final_body,
        grid=(_TGRID,),
        in_specs=[
            pl.BlockSpec((_TBLK, _D), row),
            pl.BlockSpec((_TBLK, _D), row),
            *wspecs,
        ],
        out_specs=pl.BlockSpec((_TBLK, _D), row),
        out_shape=jax.ShapeDtypeStruct((_NP, _D), jnp.float32),
    )(agg, x, *ws)


def _mlp_body(pool_ref, w1, b1, w2, b2, w3, b3, out_ref):
    p = pool_ref[...]
    x = p[0:64, :] + p[64:128, :]
    dot = functools.partial(jnp.dot, preferred_element_type=jnp.float32)
    h = _elu(dot(x, w1[...]) + b1[...])
    h = _elu(dot(h, w2[...]) + b2[...])
    y = dot(h, w3[...]) + b3[...]
    out_ref[...] = y[0:_G, :]


def _mlp_call(pool2, fc1_w, fc1_b, fc2_w, fc2_b, fc3_w, fc3_b):
    return pl.pallas_call(
        _mlp_body,
        out_shape=jax.ShapeDtypeStruct((_G, 1), jnp.float32),
    )(pool2, fc1_w, fc1_b.reshape(1, 32), fc2_w, fc2_b.reshape(1, 16),
      fc3_w, fc3_b.reshape(1, 1))


def _agg_body(m_hbm, srcp_hbm, ld0_hbm, ld1_hbm, zeros_hbm, out_hbm,
              sidx_v, didx_v, rows_v, agg_sh, sem):
    c = lax.axis_index("c")
    t = lax.axis_index("s")
    pltpu.sync_copy(zeros_hbm, agg_sh.at[pl.ds(t * _TPR, _TPR)])
    plsc.subcore_barrier()
    base0 = t * _EPT
    trips = jnp.where(t == 15, 385, 391)

    def body(i, carry):
        base = base0 + i * _CHUNK
        pltpu.sync_copy(srcp_hbm.at[pl.ds(base, _CHUNK)], sidx_v)

        @pl.when(c == 0)
        def _():
            pltpu.sync_copy(ld0_hbm.at[pl.ds(base, _CHUNK)], didx_v)

        @pl.when(c == 1)
        def _():
            pltpu.sync_copy(ld1_hbm.at[pl.ds(base, _CHUNK)], didx_v)

        pltpu.async_copy(m_hbm.at[sidx_v], rows_v, sem).wait()
        pltpu.sync_copy(rows_v, agg_sh.at[didx_v], add=True)
        return carry

    lax.fori_loop(0, trips, body, 0)
    plsc.subcore_barrier()
    pltpu.sync_copy(agg_sh.at[pl.ds(t * _TPR, _TPR)],
                    out_hbm.at[pl.ds(c * _RPAD + t * _TPR, _TPR)])


def _agg_call(m, srcp, ld0, ld1, zeros_slab):
    mesh = plsc.VectorSubcoreMesh(core_axis_name="c", subcore_axis_name="s")
    f = pl.kernel(
        _agg_body,
        out_type=jax.ShapeDtypeStruct((_NP, _D), jnp.float32),
        mesh=mesh,
        compiler_params=pltpu.CompilerParams(use_tc_tiling_on_sc=False),
        scratch_types=[
            pltpu.VMEM((_CHUNK,), jnp.int32),
            pltpu.VMEM((_CHUNK,), jnp.int32),
            pltpu.VMEM((_CHUNK, _D), jnp.float32),
            pltpu.VMEM_SHARED((_SPROWS, _D), jnp.float32),
            pltpu.SemaphoreType.DMA,
        ],
    )
    return f(m, srcp, ld0, ld1, zeros_slab)


def _pool_body(x_hbm, n2s_hbm, s2g_hbm, zeros_hbm, out_hbm,
               s2g_v, nidx_v, gidx_v, rows_v, acc_sh, sem):
    c = lax.axis_index("c")
    t = lax.axis_index("s")

    @pl.when(t == 0)
    def _():
        pltpu.sync_copy(zeros_hbm.at[pl.ds(0, 64)], acc_sh)

    pltpu.sync_copy(s2g_hbm, s2g_v)
    plsc.subcore_barrier()
    w = c * 16 + t
    base0 = w * _PPT

    def body(i, carry):
        base = base0 + i * _PBLK
        pltpu.sync_copy(x_hbm.at[pl.ds(base, _PBLK)], rows_v)
        pltpu.sync_copy(n2s_hbm.at[pl.ds(base, _PBLK)], nidx_v)
        for j in range(_PBLK // 16):
            sg = nidx_v[pl.ds(j * 16, 16)]
            gidx_v[pl.ds(j * 16, 16)] = plsc.load_gather(s2g_v, [sg])
        pltpu.sync_copy(rows_v, acc_sh.at[gidx_v], add=True)
        return carry

    lax.fori_loop(0, _PPT // _PBLK, body, 0)
    plsc.subcore_barrier()

    @pl.when(t == 0)
    def _():
        pltpu.sync_copy(acc_sh, out_hbm.at[pl.ds(c * 64, 64)])


def _pool_call(x5, n2s_p, s2g_p, zeros_slab):
    mesh = plsc.VectorSubcoreMesh(core_axis_name="c", subcore_axis_name="s")
    f = pl.kernel(
        _pool_body,
        out_type=jax.ShapeDtypeStruct((128, _D), jnp.float32),
        mesh=mesh,
        compiler_params=pltpu.CompilerParams(use_tc_tiling_on_sc=False,
                                             needs_layout_passes=False),
        scratch_types=[
            pltpu.VMEM((512,), jnp.int32),
            pltpu.VMEM((_PBLK,), jnp.int32),
            pltpu.VMEM((_PBLK,), jnp.int32),
            pltpu.VMEM((_PBLK, _D), jnp.float32),
            pltpu.VMEM_SHARED((64, _D), jnp.float32),
            pltpu.SemaphoreType.DMA,
        ],
    )
    return f(x5, n2s_p, s2g_p, zeros_slab)


def _pad_nodes(a, fill):
    pad = jnp.full((_PAD_OFF,) + a.shape[1:], fill, a.dtype)
    return jnp.concatenate([a[:_HALF], pad, a[_HALF:], pad], axis=0)


def kernel(z, edge_index, node_to_subgraph, subgraph_to_graph, node_emb, Wm,
           W_ih, W_hh, b_ih, b_hh, fc1_w, fc1_b, fc2_w, fc2_b, fc3_w, fc3_b):
    L = Wm.shape[0]
    z_p = _pad_nodes(z.astype(jnp.int32), 0).reshape(_NP, 1)
    n2s_p = _pad_nodes(node_to_subgraph.astype(jnp.int32), _S)
    s2g_p = jnp.concatenate(
        [subgraph_to_graph.astype(jnp.int32), jnp.full((12,), 63, jnp.int32)])
    zeros_slab = jnp.zeros((_TPR, _D), jnp.float32)

    srcp, ld0, ld1 = _prep_indices(edge_index[0].astype(jnp.int32),
                                   edge_index[1].astype(jnp.int32))

    x, m = _init_call(z_p, node_emb, Wm[0, 0], Wm[0, 1])
    for l in range(L):
        agg = _agg_call(m, srcp, ld0, ld1, zeros_slab)
        ws = _split_gru_weights(W_ih[l], W_hh[l], b_ih[l], b_hh[l])
        if l + 1 < L:
            x, m = _gru_msg_call(agg, x, z_p, ws, Wm[l + 1, 0], Wm[l + 1, 1])
        else:
            x = _gru_final_call(agg, x, ws)

    pool2 = _pool_call(x, n2s_p, s2g_p, zeros_slab)
    return _mlp_call(pool2, fc1_w, fc1_b, fc2_w, fc2_b, fc3_w, fc3_b)


# speedup vs baseline: 6.2307x; 1.7888x over previous
import functools

import jax
import jax.numpy as jnp
from jax import lax
from jax.experimental import pallas as pl
from jax.experimental.pallas import tpu as pltpu
import jax.experimental.pallas.tpu_sc as plsc

_N = 50000
_E = 800000
_S = 500
_G = 50
_D = 64
_H = 32

_NP = 51200
_TRASH = _NP
_SPROWS = _NP + 16
_ZROWS = _NP // 16

_CHUNK = 128
_GRP = 3
_GEDGE = _GRP * _CHUNK
_GPT = 132
_EPAD = 16 * _GPT * _GEDGE
_ECHUNKROWS = _EPAD // _CHUNK

_PBLK = 64
_PPT = _NP // 32


def _sigmoid(x):
    return 1.0 / (1.0 + jnp.exp(-x))


def _elu(x):
    return jnp.where(x > 0, x, jnp.exp(jnp.minimum(x, 0.0)) - 1.0)


_TBLK = 1024
_TGRID = _NP // _TBLK


def _init_body(z_ref, e_ref, w0_ref, w1_ref, x_ref, ma_ref, mb_ref):
    e = e_ref[...]
    x_ref[...] = jnp.broadcast_to(e, x_ref.shape)
    m0 = jnp.dot(e, w0_ref[...], preferred_element_type=jnp.float32)
    m1 = jnp.dot(e, w1_ref[...], preferred_element_type=jnp.float32)
    zf = z_ref[...] == 1
    m = jnp.where(zf, m1, m0)
    ma_ref[...] = m[:, 0:_H]
    mb_ref[...] = m[:, _H:_D]


def _init_call(z_p, node_emb, w0, w1):
    row = lambda i: (i, 0)
    fix = lambda i: (0, 0)
    return pl.pallas_call(
        _init_body,
        grid=(_TGRID,),
        in_specs=[
            pl.BlockSpec((_TBLK, 1), row),
            pl.BlockSpec((1, _D), fix),
            pl.BlockSpec((_D, _D), fix),
            pl.BlockSpec((_D, _D), fix),
        ],
        out_specs=[
            pl.BlockSpec((_TBLK, _D), row),
            pl.BlockSpec((_TBLK, _H), row),
            pl.BlockSpec((_TBLK, _H), row),
        ],
        out_shape=[jax.ShapeDtypeStruct((_NP, _D), jnp.float32),
                   jax.ShapeDtypeStruct((_NP, _H), jnp.float32),
                   jax.ShapeDtypeStruct((_NP, _H), jnp.float32)],
    )(z_p, node_emb, w0, w1)


def _gru_core(a, h, ws):
    wir, wiz, win, whr, whz, whn, bir, biz, bin_, bhr, bhz, bhn = ws
    dot = functools.partial(jnp.dot, preferred_element_type=jnp.float32)
    r = _sigmoid(dot(a, wir) + bir + dot(h, whr) + bhr)
    zg = _sigmoid(dot(a, wiz) + biz + dot(h, whz) + bhz)
    n = jnp.tanh(dot(a, win) + bin_ + r * (dot(h, whn) + bhn))
    return (1.0 - zg) * n + zg * h


def _gru_msg_body(agga_ref, aggb_ref, x_ref, z_ref, *refs):
    ws = [r[...] for r in refs[:12]]
    w0, w1 = refs[12], refs[13]
    x_out, ma_out, mb_out = refs[14], refs[15], refs[16]
    a = jnp.concatenate([agga_ref[...], aggb_ref[...]], axis=1)
    x = _gru_core(a, x_ref[...], ws)
    x_out[...] = x
    dot = functools.partial(jnp.dot, preferred_element_type=jnp.float32)
    m0 = dot(x, w0[...])
    m1 = dot(x, w1[...])
    m = jnp.where(z_ref[...] == 1, m1, m0)
    ma_out[...] = m[:, 0:_H]
    mb_out[...] = m[:, _H:_D]


def _gru_final_body(agga_ref, aggb_ref, x_ref, *refs):
    ws = [r[...] for r in refs[:12]]
    x_out = refs[12]
    a = jnp.concatenate([agga_ref[...], aggb_ref[...]], axis=1)
    x_out[...] = _gru_core(a, x_ref[...], ws)


def _split_gru_weights(W_ih_l, W_hh_l, b_ih_l, b_hh_l):
    out = []
    for w in (W_ih_l, W_hh_l):
        out += [w[:, 0:_D], w[:, _D:2 * _D], w[:, 2 * _D:3 * _D]]
    for b in (b_ih_l, b_hh_l):
        out += [b[0:_D].reshape(1, _D), b[_D:2 * _D].reshape(1, _D),
                b[2 * _D:3 * _D].reshape(1, _D)]
    return out


_ROW = lambda i: (i, 0)
_FIX = lambda i: (0, 0)
_WSPECS = ([pl.BlockSpec((_D, _D), _FIX)] * 6
           + [pl.BlockSpec((1, _D), _FIX)] * 6)


def _gru_msg_call(agga, aggb, x, z_p, ws, w0, w1):
    return pl.pallas_call(
        _gru_msg_body,
        grid=(_TGRID,),
        in_specs=[
            pl.BlockSpec((_TBLK, _H), _ROW),
            pl.BlockSpec((_TBLK, _H), _ROW),
            pl.BlockSpec((_TBLK, _D), _ROW),
            pl.BlockSpec((_TBLK, 1), _ROW),
            *_WSPECS,
            pl.BlockSpec((_D, _D), _FIX),
            pl.BlockSpec((_D, _D), _FIX),
        ],
        out_specs=[
            pl.BlockSpec((_TBLK, _D), _ROW),
            pl.BlockSpec((_TBLK, _H), _ROW),
            pl.BlockSpec((_TBLK, _H), _ROW),
        ],
        out_shape=[jax.ShapeDtypeStruct((_NP, _D), jnp.float32),
                   jax.ShapeDtypeStruct((_NP, _H), jnp.float32),
                   jax.ShapeDtypeStruct((_NP, _H), jnp.float32)],
    )(agga, aggb, x, z_p, *ws, w0, w1)


def _gru_final_call(agga, aggb, x, ws):
    return pl.pallas_call(
        _gru_final_body,
        grid=(_TGRID,),
        in_specs=[
            pl.BlockSpec((_TBLK, _H), _ROW),
            pl.BlockSpec((_TBLK, _H), _ROW),
            pl.BlockSpec((_TBLK, _D), _ROW),
            *_WSPECS,
        ],
        out_specs=pl.BlockSpec((_TBLK, _D), _ROW),
        out_shape=jax.ShapeDtypeStruct((_NP, _D), jnp.float32),
    )(agga, aggb, x, *ws)


def _mlp_body(pool_ref, w1, b1, w2, b2, w3, b3, out_ref):
    p = pool_ref[...]
    x = p[0:64, :] + p[64:128, :]
    dot = functools.partial(jnp.dot, preferred_element_type=jnp.float32)
    h = _elu(dot(x, w1[...]) + b1[...])
    h = _elu(dot(h, w2[...]) + b2[...])
    y = dot(h, w3[...]) + b3[...]
    out_ref[...] = y[0:_G, :]


def _mlp_call(pool2, fc1_w, fc1_b, fc2_w, fc2_b, fc3_w, fc3_b):
    return pl.pallas_call(
        _mlp_body,
        out_shape=jax.ShapeDtypeStruct((_G, 1), jnp.float32),
    )(pool2, fc1_w, fc1_b.reshape(1, 32), fc2_w, fc2_b.reshape(1, 16),
      fc3_w, fc3_b.reshape(1, 1))


def _agg_body(ma_hbm, mb_hbm, srcp_hbm, ldst_hbm, zeros_hbm,
              outa_hbm, outb_hbm,
              sidx0, sidx1, didx0, didx1, rows0, rows1, acc_sh,
              gsem0, gsem1, ssem0, ssem1):
    c = lax.axis_index("c")
    t = lax.axis_index("s")
    sidx = (sidx0, sidx1)
    didx = (didx0, didx1)
    rows = (rows0, rows1)
    gsem = (gsem0, gsem1)
    ssem = (ssem0, ssem1)

    pltpu.sync_copy(zeros_hbm, acc_sh.at[pl.ds(t * _ZROWS, _ZROWS)])

    @pl.when(t == 0)
    def _():
        pltpu.sync_copy(zeros_hbm.at[pl.ds(0, 16)], acc_sh.at[pl.ds(_NP, 16)])

    plsc.subcore_barrier()

    row0 = t * (_GPT * _GRP)

    def load_idx(g, b):
        r = row0 + g * _GRP
        pltpu.sync_copy(srcp_hbm.at[pl.ds(r, _GRP)], sidx[b])
        pltpu.sync_copy(ldst_hbm.at[pl.ds(r, _GRP)], didx[b])

    def fire_gathers(b):
        for j in range(_GRP):
            dst = rows[b].at[pl.ds(j * _CHUNK, _CHUNK)]

            @pl.when(c == 0)
            def _():
                pltpu.async_copy(ma_hbm.at[sidx[b].at[j]], dst, gsem[b])

            @pl.when(c == 1)
            def _():
                pltpu.async_copy(mb_hbm.at[sidx[b].at[j]], dst, gsem[b])

    def fire_scatters(b):
        for j in range(_GRP):
            src = rows[b].at[pl.ds(j * _CHUNK, _CHUNK)]
            pltpu.async_copy(src, acc_sh.at[didx[b].at[j]], ssem[b], add=True)

    def drain(sem, b):
        pltpu.make_async_copy(ma_hbm.at[pl.ds(0, _GEDGE)], rows[b], sem).wait()

    load_idx(0, 0)
    fire_gathers(0)

    def pair_body(p, carry):
        for b in (0, 1):
            g = 2 * p + b
            drain(gsem[b], b)
            fire_scatters(b)

            @pl.when(g > 0)
            def _():
                drain(ssem[1 - b], 1 - b)

            @pl.when(g + 1 < _GPT)
            def _():
                load_idx(g + 1, 1 - b)
                fire_gathers(1 - b)
        return carry

    lax.fori_loop(0, _GPT // 2, pair_body, 0)
    drain(ssem[(_GPT - 1) % 2], (_GPT - 1) % 2)

    plsc.subcore_barrier()
    sl = pl.ds(t * _ZROWS, _ZROWS)

    @pl.when(c == 0)
    def _():
        pltpu.sync_copy(acc_sh.at[sl], outa_hbm.at[sl])

    @pl.when(c == 1)
    def _():
        pltpu.sync_copy(acc_sh.at[sl], outb_hbm.at[sl])


def _agg_call(ma, mb, srcp2, ldst2, zeros_agg):
    mesh = plsc.VectorSubcoreMesh(core_axis_name="c", subcore_axis_name="s")
    f = pl.kernel(
        _agg_body,
        out_type=[jax.ShapeDtypeStruct((_NP, _H), jnp.float32),
                  jax.ShapeDtypeStruct((_NP, _H), jnp.float32)],
        mesh=mesh,
        compiler_params=pltpu.CompilerParams(use_tc_tiling_on_sc=False),
        scratch_types=[
            pltpu.VMEM((_GRP, _CHUNK), jnp.int32),
            pltpu.VMEM((_GRP, _CHUNK), jnp.int32),
            pltpu.VMEM((_GRP, _CHUNK), jnp.int32),
            pltpu.VMEM((_GRP, _CHUNK), jnp.int32),
            pltpu.VMEM((_GEDGE, _H), jnp.float32),
            pltpu.VMEM((_GEDGE, _H), jnp.float32),
            pltpu.VMEM_SHARED((_SPROWS, _H), jnp.float32),
            pltpu.SemaphoreType.DMA,
            pltpu.SemaphoreType.DMA,
            pltpu.SemaphoreType.DMA,
            pltpu.SemaphoreType.DMA,
        ],
    )
    return f(ma, mb, srcp2, ldst2, zeros_agg)


def _pool_body(x_hbm, n2s_hbm, s2g_hbm, zeros_hbm, out_hbm,
               s2g_v, nidx_v, gidx_v, rows_v, acc_sh, sem):
    c = lax.axis_index("c")
    t = lax.axis_index("s")

    @pl.when(t == 0)
    def _():
        pltpu.sync_copy(zeros_hbm, acc_sh)

    pltpu.sync_copy(s2g_hbm, s2g_v)
    plsc.subcore_barrier()
    w = c * 16 + t
    base0 = w * _PPT

    def body(i, carry):
        base = base0 + i * _PBLK
        pltpu.sync_copy(x_hbm.at[pl.ds(base, _PBLK)], rows_v)
        pltpu.sync_copy(n2s_hbm.at[pl.ds(base, _PBLK)], nidx_v)
        for j in range(_PBLK // 16):
            sg = nidx_v[pl.ds(j * 16, 16)]
            gidx_v[pl.ds(j * 16, 16)] = plsc.load_gather(s2g_v, [sg])
        pltpu.sync_copy(rows_v, acc_sh.at[gidx_v], add=True)
        return carry

    lax.fori_loop(0, _PPT // _PBLK, body, 0)
    plsc.subcore_barrier()

    @pl.when(t == 0)
    def _():
        pltpu.sync_copy(acc_sh, out_hbm.at[pl.ds(c * 64, 64)])


def _pool_call(x5, n2s_p, s2g_p, zeros_pool):
    mesh = plsc.VectorSubcoreMesh(core_axis_name="c", subcore_axis_name="s")
    f = pl.kernel(
        _pool_body,
        out_type=jax.ShapeDtypeStruct((128, _D), jnp.float32),
        mesh=mesh,
        compiler_params=pltpu.CompilerParams(use_tc_tiling_on_sc=False,
                                             needs_layout_passes=False),
        scratch_types=[
            pltpu.VMEM((512,), jnp.int32),
            pltpu.VMEM((_PBLK,), jnp.int32),
            pltpu.VMEM((_PBLK,), jnp.int32),
            pltpu.VMEM((_PBLK, _D), jnp.float32),
            pltpu.VMEM_SHARED((64, _D), jnp.float32),
            pltpu.SemaphoreType.DMA,
        ],
    )
    return f(x5, n2s_p, s2g_p, zeros_pool)


def kernel(z, edge_index, node_to_subgraph, subgraph_to_graph, node_emb, Wm,
           W_ih, W_hh, b_ih, b_hh, fc1_w, fc1_b, fc2_w, fc2_b, fc3_w, fc3_b):
    L = Wm.shape[0]
    npad = _NP - _N
    z_p = jnp.concatenate(
        [z.astype(jnp.int32), jnp.zeros((npad,), jnp.int32)]).reshape(_NP, 1)
    n2s_p = jnp.concatenate(
        [node_to_subgraph.astype(jnp.int32), jnp.full((npad,), _S, jnp.int32)])
    s2g_p = jnp.concatenate(
        [subgraph_to_graph.astype(jnp.int32), jnp.full((12,), 63, jnp.int32)])
    zeros_agg = jnp.zeros((_ZROWS, _H), jnp.float32)
    zeros_pool = jnp.zeros((64, _D), jnp.float32)

    epad = _EPAD - _E
    srcp2 = jnp.concatenate(
        [edge_index[0].astype(jnp.int32), jnp.zeros((epad,), jnp.int32)]
    ).reshape(_ECHUNKROWS, _CHUNK)
    ldst2 = jnp.concatenate(
        [edge_index[1].astype(jnp.int32), jnp.full((epad,), _TRASH, jnp.int32)]
    ).reshape(_ECHUNKROWS, _CHUNK)

    x, ma, mb = _init_call(z_p, node_emb, Wm[0, 0], Wm[0, 1])
    for l in range(L):
        agga, aggb = _agg_call(ma, mb, srcp2, ldst2, zeros_agg)
        ws = _split_gru_weights(W_ih[l], W_hh[l], b_ih[l], b_hh[l])
        if l + 1 < L:
            x, ma, mb = _gru_msg_call(agga, aggb, x, z_p, ws,
                                      Wm[l + 1, 0], Wm[l + 1, 1])
        else:
            x = _gru_final_call(agga, aggb, x, ws)

    pool2 = _pool_call(x, n2s_p, s2g_p, zeros_pool)
    return _mlp_call(pool2, fc1_w, fc1_b, fc2_w, fc2_b, fc3_w, fc3_b)


# speedup vs baseline: 6.2329x; 1.0004x over previous
import functools

import jax
import jax.numpy as jnp
from jax import lax
from jax.experimental import pallas as pl
from jax.experimental.pallas import tpu as pltpu
import jax.experimental.pallas.tpu_sc as plsc

_N = 50000
_E = 800000
_S = 500
_G = 50
_D = 64
_H = 32

_NP = 51200
_TRASH = _NP
_SPROWS = _NP + 16
_ZROWS = _NP // 16

_CHUNK = 128
_GRP = 3
_GEDGE = _GRP * _CHUNK
_GPT = 132
_EPAD = 16 * _GPT * _GEDGE
_ECHUNKROWS = _EPAD // _CHUNK

_PBLK = 64
_PPT = _NP // 32


def _sigmoid(x):
    return 1.0 / (1.0 + jnp.exp(-x))


def _elu(x):
    return jnp.where(x > 0, x, jnp.exp(jnp.minimum(x, 0.0)) - 1.0)


_TBLK = 1024
_TGRID = _NP // _TBLK


def _init_body(z_ref, e_ref, w0_ref, w1_ref, x_ref, ma_ref, mb_ref):
    e = e_ref[...]
    x_ref[...] = jnp.broadcast_to(e, x_ref.shape)
    m0 = jnp.dot(e, w0_ref[...], preferred_element_type=jnp.float32)
    m1 = jnp.dot(e, w1_ref[...], preferred_element_type=jnp.float32)
    zf = z_ref[...] == 1
    m = jnp.where(zf, m1, m0)
    ma_ref[...] = m[:, 0:_H]
    mb_ref[...] = m[:, _H:_D]


def _init_call(z_p, node_emb, w0, w1):
    row = lambda i: (i, 0)
    fix = lambda i: (0, 0)
    return pl.pallas_call(
        _init_body,
        grid=(_TGRID,),
        in_specs=[
            pl.BlockSpec((_TBLK, 1), row),
            pl.BlockSpec((1, _D), fix),
            pl.BlockSpec((_D, _D), fix),
            pl.BlockSpec((_D, _D), fix),
        ],
        out_specs=[
            pl.BlockSpec((_TBLK, _D), row),
            pl.BlockSpec((_TBLK, _H), row),
            pl.BlockSpec((_TBLK, _H), row),
        ],
        out_shape=[jax.ShapeDtypeStruct((_NP, _D), jnp.float32),
                   jax.ShapeDtypeStruct((_NP, _H), jnp.float32),
                   jax.ShapeDtypeStruct((_NP, _H), jnp.float32)],
    )(z_p, node_emb, w0, w1)


def _gru_core(a, h, ws):
    wir, wiz, win, whr, whz, whn, bir, biz, bin_, bhr, bhz, bhn = ws
    dot = functools.partial(jnp.dot, preferred_element_type=jnp.float32)
    r = _sigmoid(dot(a, wir) + bir + dot(h, whr) + bhr)
    zg = _sigmoid(dot(a, wiz) + biz + dot(h, whz) + bhz)
    n = jnp.tanh(dot(a, win) + bin_ + r * (dot(h, whn) + bhn))
    return (1.0 - zg) * n + zg * h


def _gru_msg_body(agga_ref, aggb_ref, x_ref, z_ref, *refs):
    ws = [r[...] for r in refs[:12]]
    w0, w1 = refs[12], refs[13]
    x_out, ma_out, mb_out = refs[14], refs[15], refs[16]
    a = jnp.concatenate([agga_ref[...], aggb_ref[...]], axis=1)
    x = _gru_core(a, x_ref[...], ws)
    x_out[...] = x
    dot = functools.partial(jnp.dot, preferred_element_type=jnp.float32)
    m0 = dot(x, w0[...])
    m1 = dot(x, w1[...])
    m = jnp.where(z_ref[...] == 1, m1, m0)
    ma_out[...] = m[:, 0:_H]
    mb_out[...] = m[:, _H:_D]


def _gru_final_body(agga_ref, aggb_ref, x_ref, *refs):
    ws = [r[...] for r in refs[:12]]
    x_out = refs[12]
    a = jnp.concatenate([agga_ref[...], aggb_ref[...]], axis=1)
    x_out[...] = _gru_core(a, x_ref[...], ws)


def _split_gru_weights(W_ih_l, W_hh_l, b_ih_l, b_hh_l):
    out = []
    for w in (W_ih_l, W_hh_l):
        out += [w[:, 0:_D], w[:, _D:2 * _D], w[:, 2 * _D:3 * _D]]
    for b in (b_ih_l, b_hh_l):
        out += [b[0:_D].reshape(1, _D), b[_D:2 * _D].reshape(1, _D),
                b[2 * _D:3 * _D].reshape(1, _D)]
    return out


_ROW = lambda i: (i, 0)
_FIX = lambda i: (0, 0)
_WSPECS = ([pl.BlockSpec((_D, _D), _FIX)] * 6
           + [pl.BlockSpec((1, _D), _FIX)] * 6)


def _gru_msg_call(agga, aggb, x, z_p, ws, w0, w1):
    return pl.pallas_call(
        _gru_msg_body,
        grid=(_TGRID,),
        in_specs=[
            pl.BlockSpec((_TBLK, _H), _ROW),
            pl.BlockSpec((_TBLK, _H), _ROW),
            pl.BlockSpec((_TBLK, _D), _ROW),
            pl.BlockSpec((_TBLK, 1), _ROW),
            *_WSPECS,
            pl.BlockSpec((_D, _D), _FIX),
            pl.BlockSpec((_D, _D), _FIX),
        ],
        out_specs=[
            pl.BlockSpec((_TBLK, _D), _ROW),
            pl.BlockSpec((_TBLK, _H), _ROW),
            pl.BlockSpec((_TBLK, _H), _ROW),
        ],
        out_shape=[jax.ShapeDtypeStruct((_NP, _D), jnp.float32),
                   jax.ShapeDtypeStruct((_NP, _H), jnp.float32),
                   jax.ShapeDtypeStruct((_NP, _H), jnp.float32)],
    )(agga, aggb, x, z_p, *ws, w0, w1)


def _gru_final_call(agga, aggb, x, ws):
    return pl.pallas_call(
        _gru_final_body,
        grid=(_TGRID,),
        in_specs=[
            pl.BlockSpec((_TBLK, _H), _ROW),
            pl.BlockSpec((_TBLK, _H), _ROW),
            pl.BlockSpec((_TBLK, _D), _ROW),
            *_WSPECS,
        ],
        out_specs=pl.BlockSpec((_TBLK, _D), _ROW),
        out_shape=jax.ShapeDtypeStruct((_NP, _D), jnp.float32),
    )(agga, aggb, x, *ws)


def _mlp_body(pool_ref, w1, b1, w2, b2, w3, b3, out_ref):
    p = pool_ref[...]
    x = p[0:64, :] + p[64:128, :]
    dot = functools.partial(jnp.dot, preferred_element_type=jnp.float32)
    h = _elu(dot(x, w1[...]) + b1[...])
    h = _elu(dot(h, w2[...]) + b2[...])
    y = dot(h, w3[...]) + b3[...]
    out_ref[...] = y[0:_G, :]


def _mlp_call(pool2, fc1_w, fc1_b, fc2_w, fc2_b, fc3_w, fc3_b):
    return pl.pallas_call(
        _mlp_body,
        out_shape=jax.ShapeDtypeStruct((_G, 1), jnp.float32),
    )(pool2, fc1_w, fc1_b.reshape(1, 32), fc2_w, fc2_b.reshape(1, 16),
      fc3_w, fc3_b.reshape(1, 1))


def _agg_body(ma_hbm, mb_hbm, srcp_hbm, ldst_hbm, zeros_hbm,
              outa_hbm, outb_hbm,
              sidx0, sidx1, didx0, didx1, rows0, rows1, acc_sh,
              gsem0, gsem1, ssem0, ssem1):
    c = lax.axis_index("c")
    t = lax.axis_index("s")
    sidx = (sidx0, sidx1)
    didx = (didx0, didx1)
    rows = (rows0, rows1)
    gsem = (gsem0, gsem1)
    ssem = (ssem0, ssem1)

    pltpu.sync_copy(zeros_hbm, acc_sh.at[pl.ds(t * _ZROWS, _ZROWS)])

    @pl.when(t == 0)
    def _():
        pltpu.sync_copy(zeros_hbm.at[pl.ds(0, 16)], acc_sh.at[pl.ds(_NP, 16)])

    plsc.subcore_barrier()

    row0 = t * (_GPT * _GRP)

    def load_idx(g, b):
        e0 = (row0 + g * _GRP) * _CHUNK
        pltpu.sync_copy(srcp_hbm.at[pl.ds(e0, _GEDGE)], sidx[b])
        pltpu.sync_copy(ldst_hbm.at[pl.ds(e0, _GEDGE)], didx[b])

    def fire_gathers(b):
        @pl.when(c == 0)
        def _():
            pltpu.async_copy(ma_hbm.at[sidx[b]], rows[b], gsem[b])

        @pl.when(c == 1)
        def _():
            pltpu.async_copy(mb_hbm.at[sidx[b]], rows[b], gsem[b])

    def fire_scatters(b):
        pltpu.async_copy(rows[b], acc_sh.at[didx[b]], ssem[b], add=True)

    def drain(sem, b):
        pltpu.make_async_copy(ma_hbm.at[pl.ds(0, _GEDGE)], rows[b], sem).wait()

    load_idx(0, 0)
    fire_gathers(0)

    def pair_body(p, carry):
        for b in (0, 1):
            g = 2 * p + b
            drain(gsem[b], b)
            fire_scatters(b)

            @pl.when(g > 0)
            def _():
                drain(ssem[1 - b], 1 - b)

            @pl.when(g + 1 < _GPT)
            def _():
                load_idx(g + 1, 1 - b)
                fire_gathers(1 - b)
        return carry

    lax.fori_loop(0, _GPT // 2, pair_body, 0)
    drain(ssem[(_GPT - 1) % 2], (_GPT - 1) % 2)

    plsc.subcore_barrier()
    sl = pl.ds(t * _ZROWS, _ZROWS)

    @pl.when(c == 0)
    def _():
        pltpu.sync_copy(acc_sh.at[sl], outa_hbm.at[sl])

    @pl.when(c == 1)
    def _():
        pltpu.sync_copy(acc_sh.at[sl], outb_hbm.at[sl])


def _agg_call(ma, mb, srcp2, ldst2, zeros_agg):
    mesh = plsc.VectorSubcoreMesh(core_axis_name="c", subcore_axis_name="s")
    f = pl.kernel(
        _agg_body,
        out_type=[jax.ShapeDtypeStruct((_NP, _H), jnp.float32),
                  jax.ShapeDtypeStruct((_NP, _H), jnp.float32)],
        mesh=mesh,
        compiler_params=pltpu.CompilerParams(use_tc_tiling_on_sc=False),
        scratch_types=[
            pltpu.VMEM((_GEDGE,), jnp.int32),
            pltpu.VMEM((_GEDGE,), jnp.int32),
            pltpu.VMEM((_GEDGE,), jnp.int32),
            pltpu.VMEM((_GEDGE,), jnp.int32),
            pltpu.VMEM((_GEDGE, _H), jnp.float32),
            pltpu.VMEM((_GEDGE, _H), jnp.float32),
            pltpu.VMEM_SHARED((_SPROWS, _H), jnp.float32),
            pltpu.SemaphoreType.DMA,
            pltpu.SemaphoreType.DMA,
            pltpu.SemaphoreType.DMA,
            pltpu.SemaphoreType.DMA,
        ],
    )
    return f(ma, mb, srcp2, ldst2, zeros_agg)


def _pool_body(x_hbm, n2s_hbm, s2g_hbm, zeros_hbm, out_hbm,
               s2g_v, nidx_v, gidx_v, rows_v, acc_sh, sem):
    c = lax.axis_index("c")
    t = lax.axis_index("s")

    @pl.when(t == 0)
    def _():
        pltpu.sync_copy(zeros_hbm, acc_sh)

    pltpu.sync_copy(s2g_hbm, s2g_v)
    plsc.subcore_barrier()
    w = c * 16 + t
    base0 = w * _PPT

    def body(i, carry):
        base = base0 + i * _PBLK
        pltpu.sync_copy(x_hbm.at[pl.ds(base, _PBLK)], rows_v)
        pltpu.sync_copy(n2s_hbm.at[pl.ds(base, _PBLK)], nidx_v)
        for j in range(_PBLK // 16):
            sg = nidx_v[pl.ds(j * 16, 16)]
            gidx_v[pl.ds(j * 16, 16)] = plsc.load_gather(s2g_v, [sg])
        pltpu.sync_copy(rows_v, acc_sh.at[gidx_v], add=True)
        return carry

    lax.fori_loop(0, _PPT // _PBLK, body, 0)
    plsc.subcore_barrier()

    @pl.when(t == 0)
    def _():
        pltpu.sync_copy(acc_sh, out_hbm.at[pl.ds(c * 64, 64)])


def _pool_call(x5, n2s_p, s2g_p, zeros_pool):
    mesh = plsc.VectorSubcoreMesh(core_axis_name="c", subcore_axis_name="s")
    f = pl.kernel(
        _pool_body,
        out_type=jax.ShapeDtypeStruct((128, _D), jnp.float32),
        mesh=mesh,
        compiler_params=pltpu.CompilerParams(use_tc_tiling_on_sc=False,
                                             needs_layout_passes=False),
        scratch_types=[
            pltpu.VMEM((512,), jnp.int32),
            pltpu.VMEM((_PBLK,), jnp.int32),
            pltpu.VMEM((_PBLK,), jnp.int32),
            pltpu.VMEM((_PBLK, _D), jnp.float32),
            pltpu.VMEM_SHARED((64, _D), jnp.float32),
            pltpu.SemaphoreType.DMA,
        ],
    )
    return f(x5, n2s_p, s2g_p, zeros_pool)


def kernel(z, edge_index, node_to_subgraph, subgraph_to_graph, node_emb, Wm,
           W_ih, W_hh, b_ih, b_hh, fc1_w, fc1_b, fc2_w, fc2_b, fc3_w, fc3_b):
    L = Wm.shape[0]
    npad = _NP - _N
    z_p = jnp.concatenate(
        [z.astype(jnp.int32), jnp.zeros((npad,), jnp.int32)]).reshape(_NP, 1)
    n2s_p = jnp.concatenate(
        [node_to_subgraph.astype(jnp.int32), jnp.full((npad,), _S, jnp.int32)])
    s2g_p = jnp.concatenate(
        [subgraph_to_graph.astype(jnp.int32), jnp.full((12,), 63, jnp.int32)])
    zeros_agg = jnp.zeros((_ZROWS, _H), jnp.float32)
    zeros_pool = jnp.zeros((64, _D), jnp.float32)

    epad = _EPAD - _E
    srcp2 = jnp.concatenate(
        [edge_index[0].astype(jnp.int32), jnp.zeros((epad,), jnp.int32)])
    ldst2 = jnp.concatenate(
        [edge_index[1].astype(jnp.int32), jnp.full((epad,), _TRASH, jnp.int32)])

    x, ma, mb = _init_call(z_p, node_emb, Wm[0, 0], Wm[0, 1])
    for l in range(L):
        agga, aggb = _agg_call(ma, mb, srcp2, ldst2, zeros_agg)
        ws = _split_gru_weights(W_ih[l], W_hh[l], b_ih[l], b_hh[l])
        if l + 1 < L:
            x, ma, mb = _gru_msg_call(agga, aggb, x, z_p, ws,
                                      Wm[l + 1, 0], Wm[l + 1, 1])
        else:
            x = _gru_final_call(agga, aggb, x, ws)

    pool2 = _pool_call(x, n2s_p, s2g_p, zeros_pool)
    return _mlp_call(pool2, fc1_w, fc1_b, fc2_w, fc2_b, fc3_w, fc3_b)


# speedup vs baseline: 6.4793x; 1.0395x over previous
import functools

import jax
import jax.numpy as jnp
from jax import lax
from jax.experimental import pallas as pl
from jax.experimental.pallas import tpu as pltpu
import jax.experimental.pallas.tpu_sc as plsc

_N = 50000
_E = 800000
_S = 500
_G = 50
_D = 64
_H = 32

_NP = 51200
_TRASH = _NP
_SPROWS = _NP + 16
_ZROWS = _NP // 16

_CHUNK = 128
_GRP = 3
_GEDGE = _GRP * _CHUNK
_GPT = 132
_EPAD = 16 * _GPT * _GEDGE
_ECHUNKROWS = _EPAD // _CHUNK

_CW = 8
_GPT0 = _EPAD // (32 * _GEDGE)

_PBLK = 64
_PPT = _NP // 32


def _sigmoid(x):
    return 1.0 / (1.0 + jnp.exp(-x))


def _elu(x):
    return jnp.where(x > 0, x, jnp.exp(jnp.minimum(x, 0.0)) - 1.0)


_TBLK = 1024
_TGRID = _NP // _TBLK


def _gru0_body(acca_ref, accb_ref, z_ref, e_ref, wm00, wm01, *refs):
    ws = [r[...] for r in refs[:12]]
    w0, w1 = refs[12], refs[13]
    x_out, ma_out, mb_out = refs[14], refs[15], refs[16]
    acc = acca_ref[...] + accb_ref[...]
    cnt = acc[:, 0:1]
    sz = acc[:, 1:2]
    e = e_ref[...]
    dot = functools.partial(jnp.dot, preferred_element_type=jnp.float32)
    m0v = dot(e, wm00[...])
    m1v = dot(e, wm01[...])
    a = (cnt - sz) * m0v + sz * m1v
    h = jnp.broadcast_to(e, a.shape)
    x = _gru_core(a, h, ws)
    x_out[...] = x
    m0 = dot(x, w0[...])
    m1 = dot(x, w1[...])
    m = jnp.where(z_ref[...] == 1, m1, m0)
    ma_out[...] = m[:, 0:_H]
    mb_out[...] = m[:, _H:_D]


def _gru0_call(acca, accb, z_p, node_emb, wm00, wm01, ws, w0, w1):
    return pl.pallas_call(
        _gru0_body,
        grid=(_TGRID,),
        in_specs=[
            pl.BlockSpec((_TBLK, _CW), _ROW),
            pl.BlockSpec((_TBLK, _CW), _ROW),
            pl.BlockSpec((_TBLK, 1), _ROW),
            pl.BlockSpec((1, _D), _FIX),
            pl.BlockSpec((_D, _D), _FIX),
            pl.BlockSpec((_D, _D), _FIX),
            *_WSPECS,
            pl.BlockSpec((_D, _D), _FIX),
            pl.BlockSpec((_D, _D), _FIX),
        ],
        out_specs=[
            pl.BlockSpec((_TBLK, _D), _ROW),
            pl.BlockSpec((_TBLK, _H), _ROW),
            pl.BlockSpec((_TBLK, _H), _ROW),
        ],
        out_shape=[jax.ShapeDtypeStruct((_NP, _D), jnp.float32),
                   jax.ShapeDtypeStruct((_NP, _H), jnp.float32),
                   jax.ShapeDtypeStruct((_NP, _H), jnp.float32)],
    )(acca, accb, z_p, node_emb, wm00, wm01, *ws, w0, w1)


def _gru_core(a, h, ws):
    wir, wiz, win, whr, whz, whn, bir, biz, bin_, bhr, bhz, bhn = ws
    dot = functools.partial(jnp.dot, preferred_element_type=jnp.float32)
    r = _sigmoid(dot(a, wir) + bir + dot(h, whr) + bhr)
    zg = _sigmoid(dot(a, wiz) + biz + dot(h, whz) + bhz)
    n = jnp.tanh(dot(a, win) + bin_ + r * (dot(h, whn) + bhn))
    return (1.0 - zg) * n + zg * h


def _gru_msg_body(agga_ref, aggb_ref, x_ref, z_ref, *refs):
    ws = [r[...] for r in refs[:12]]
    w0, w1 = refs[12], refs[13]
    x_out, ma_out, mb_out = refs[14], refs[15], refs[16]
    a = jnp.concatenate([agga_ref[...], aggb_ref[...]], axis=1)
    x = _gru_core(a, x_ref[...], ws)
    x_out[...] = x
    dot = functools.partial(jnp.dot, preferred_element_type=jnp.float32)
    m0 = dot(x, w0[...])
    m1 = dot(x, w1[...])
    m = jnp.where(z_ref[...] == 1, m1, m0)
    ma_out[...] = m[:, 0:_H]
    mb_out[...] = m[:, _H:_D]


def _gru_final_body(agga_ref, aggb_ref, x_ref, *refs):
    ws = [r[...] for r in refs[:12]]
    x_out = refs[12]
    a = jnp.concatenate([agga_ref[...], aggb_ref[...]], axis=1)
    x_out[...] = _gru_core(a, x_ref[...], ws)


def _split_gru_weights(W_ih_l, W_hh_l, b_ih_l, b_hh_l):
    out = []
    for w in (W_ih_l, W_hh_l):
        out += [w[:, 0:_D], w[:, _D:2 * _D], w[:, 2 * _D:3 * _D]]
    for b in (b_ih_l, b_hh_l):
        out += [b[0:_D].reshape(1, _D), b[_D:2 * _D].reshape(1, _D),
                b[2 * _D:3 * _D].reshape(1, _D)]
    return out


_ROW = lambda i: (i, 0)
_FIX = lambda i: (0, 0)
_WSPECS = ([pl.BlockSpec((_D, _D), _FIX)] * 6
           + [pl.BlockSpec((1, _D), _FIX)] * 6)


def _gru_msg_call(agga, aggb, x, z_p, ws, w0, w1):
    return pl.pallas_call(
        _gru_msg_body,
        grid=(_TGRID,),
        in_specs=[
            pl.BlockSpec((_TBLK, _H), _ROW),
            pl.BlockSpec((_TBLK, _H), _ROW),
            pl.BlockSpec((_TBLK, _D), _ROW),
            pl.BlockSpec((_TBLK, 1), _ROW),
            *_WSPECS,
            pl.BlockSpec((_D, _D), _FIX),
            pl.BlockSpec((_D, _D), _FIX),
        ],
        out_specs=[
            pl.BlockSpec((_TBLK, _D), _ROW),
            pl.BlockSpec((_TBLK, _H), _ROW),
            pl.BlockSpec((_TBLK, _H), _ROW),
        ],
        out_shape=[jax.ShapeDtypeStruct((_NP, _D), jnp.float32),
                   jax.ShapeDtypeStruct((_NP, _H), jnp.float32),
                   jax.ShapeDtypeStruct((_NP, _H), jnp.float32)],
    )(agga, aggb, x, z_p, *ws, w0, w1)


def _gru_final_call(agga, aggb, x, ws):
    return pl.pallas_call(
        _gru_final_body,
        grid=(_TGRID,),
        in_specs=[
            pl.BlockSpec((_TBLK, _H), _ROW),
            pl.BlockSpec((_TBLK, _H), _ROW),
            pl.BlockSpec((_TBLK, _D), _ROW),
            *_WSPECS,
        ],
        out_specs=pl.BlockSpec((_TBLK, _D), _ROW),
        out_shape=jax.ShapeDtypeStruct((_NP, _D), jnp.float32),
    )(agga, aggb, x, *ws)


def _mlp_body(pool_ref, w1, b1, w2, b2, w3, b3, out_ref):
    p = pool_ref[...]
    x = p[0:64, :] + p[64:128, :]
    dot = functools.partial(jnp.dot, preferred_element_type=jnp.float32)
    h = _elu(dot(x, w1[...]) + b1[...])
    h = _elu(dot(h, w2[...]) + b2[...])
    y = dot(h, w3[...]) + b3[...]
    out_ref[...] = y[0:_G, :]


def _mlp_call(pool2, fc1_w, fc1_b, fc2_w, fc2_b, fc3_w, fc3_b):
    return pl.pallas_call(
        _mlp_body,
        out_shape=jax.ShapeDtypeStruct((_G, 1), jnp.float32),
    )(pool2, fc1_w, fc1_b.reshape(1, 32), fc2_w, fc2_b.reshape(1, 16),
      fc3_w, fc3_b.reshape(1, 1))


def _agg_body(ma_hbm, mb_hbm, srcp_hbm, ldst_hbm, zeros_hbm,
              outa_hbm, outb_hbm,
              sidx0, sidx1, didx0, didx1, rows0, rows1, acc_sh,
              gsem0, gsem1, ssem0, ssem1):
    c = lax.axis_index("c")
    t = lax.axis_index("s")
    sidx = (sidx0, sidx1)
    didx = (didx0, didx1)
    rows = (rows0, rows1)
    gsem = (gsem0, gsem1)
    ssem = (ssem0, ssem1)

    pltpu.sync_copy(zeros_hbm, acc_sh.at[pl.ds(t * _ZROWS, _ZROWS)])

    @pl.when(t == 0)
    def _():
        pltpu.sync_copy(zeros_hbm.at[pl.ds(0, 16)], acc_sh.at[pl.ds(_NP, 16)])

    plsc.subcore_barrier()

    row0 = t * (_GPT * _GRP)

    def load_idx(g, b):
        e0 = (row0 + g * _GRP) * _CHUNK
        pltpu.sync_copy(srcp_hbm.at[pl.ds(e0, _GEDGE)], sidx[b])
        pltpu.sync_copy(ldst_hbm.at[pl.ds(e0, _GEDGE)], didx[b])

    def fire_gathers(b):
        @pl.when(c == 0)
        def _():
            pltpu.async_copy(ma_hbm.at[sidx[b]], rows[b], gsem[b])

        @pl.when(c == 1)
        def _():
            pltpu.async_copy(mb_hbm.at[sidx[b]], rows[b], gsem[b])

    def fire_scatters(b):
        pltpu.async_copy(rows[b], acc_sh.at[didx[b]], ssem[b], add=True)

    def drain(sem, b):
        pltpu.make_async_copy(ma_hbm.at[pl.ds(0, _GEDGE)], rows[b], sem).wait()

    load_idx(0, 0)
    fire_gathers(0)

    def pair_body(p, carry):
        for b in (0, 1):
            g = 2 * p + b
            drain(gsem[b], b)
            fire_scatters(b)

            @pl.when(g > 0)
            def _():
                drain(ssem[1 - b], 1 - b)

            @pl.when(g + 1 < _GPT)
            def _():
                load_idx(g + 1, 1 - b)
                fire_gathers(1 - b)
        return carry

    lax.fori_loop(0, _GPT // 2, pair_body, 0)
    drain(ssem[(_GPT - 1) % 2], (_GPT - 1) % 2)

    plsc.subcore_barrier()
    sl = pl.ds(t * _ZROWS, _ZROWS)

    @pl.when(c == 0)
    def _():
        pltpu.sync_copy(acc_sh.at[sl], outa_hbm.at[sl])

    @pl.when(c == 1)
    def _():
        pltpu.sync_copy(acc_sh.at[sl], outb_hbm.at[sl])


def _agg_call(ma, mb, srcp2, ldst2, zeros_agg):
    mesh = plsc.VectorSubcoreMesh(core_axis_name="c", subcore_axis_name="s")
    f = pl.kernel(
        _agg_body,
        out_type=[jax.ShapeDtypeStruct((_NP, _H), jnp.float32),
                  jax.ShapeDtypeStruct((_NP, _H), jnp.float32)],
        mesh=mesh,
        compiler_params=pltpu.CompilerParams(use_tc_tiling_on_sc=False),
        scratch_types=[
            pltpu.VMEM((_GEDGE,), jnp.int32),
            pltpu.VMEM((_GEDGE,), jnp.int32),
            pltpu.VMEM((_GEDGE,), jnp.int32),
            pltpu.VMEM((_GEDGE,), jnp.int32),
            pltpu.VMEM((_GEDGE, _H), jnp.float32),
            pltpu.VMEM((_GEDGE, _H), jnp.float32),
            pltpu.VMEM_SHARED((_SPROWS, _H), jnp.float32),
            pltpu.SemaphoreType.DMA,
            pltpu.SemaphoreType.DMA,
            pltpu.SemaphoreType.DMA,
            pltpu.SemaphoreType.DMA,
        ],
    )
    return f(ma, mb, srcp2, ldst2, zeros_agg)


def _cnt_body(zt_hbm, srcp_hbm, ldst_hbm, zeros_hbm, outa_hbm, outb_hbm,
              sidx0, sidx1, didx0, didx1, rows0, rows1, acc_sh,
              gsem0, gsem1, ssem0, ssem1):
    c = lax.axis_index("c")
    t = lax.axis_index("s")
    sidx = (sidx0, sidx1)
    didx = (didx0, didx1)
    rows = (rows0, rows1)
    gsem = (gsem0, gsem1)
    ssem = (ssem0, ssem1)

    pltpu.sync_copy(zeros_hbm, acc_sh.at[pl.ds(t * _ZROWS, _ZROWS)])

    @pl.when(t == 0)
    def _():
        pltpu.sync_copy(zeros_hbm.at[pl.ds(0, 16)], acc_sh.at[pl.ds(_NP, 16)])

    plsc.subcore_barrier()

    base0 = (c * 16 + t) * (_GPT0 * _GEDGE)

    def load_idx(g, b):
        e0 = base0 + g * _GEDGE
        pltpu.sync_copy(srcp_hbm.at[pl.ds(e0, _GEDGE)], sidx[b])
        pltpu.sync_copy(ldst_hbm.at[pl.ds(e0, _GEDGE)], didx[b])

    def fire_gathers(b):
        pltpu.async_copy(zt_hbm.at[sidx[b]], rows[b], gsem[b])

    def fire_scatters(b):
        pltpu.async_copy(rows[b], acc_sh.at[didx[b]], ssem[b], add=True)

    def drain(sem, b):
        pltpu.make_async_copy(zt_hbm.at[pl.ds(0, _GEDGE)], rows[b], sem).wait()

    load_idx(0, 0)
    fire_gathers(0)

    def pair_body(p, carry):
        for b in (0, 1):
            g = 2 * p + b
            drain(gsem[b], b)
            fire_scatters(b)

            @pl.when(g > 0)
            def _():
                drain(ssem[1 - b], 1 - b)

            @pl.when(g + 1 < _GPT0)
            def _():
                load_idx(g + 1, 1 - b)
                fire_gathers(1 - b)
        return carry

    lax.fori_loop(0, _GPT0 // 2, pair_body, 0)
    drain(ssem[(_GPT0 - 1) % 2], (_GPT0 - 1) % 2)

    plsc.subcore_barrier()
    sl = pl.ds(t * _ZROWS, _ZROWS)

    @pl.when(c == 0)
    def _():
        pltpu.sync_copy(acc_sh.at[sl], outa_hbm.at[sl])

    @pl.when(c == 1)
    def _():
        pltpu.sync_copy(acc_sh.at[sl], outb_hbm.at[sl])


def _cnt_call(zt, srcp2, ldst2, zeros_cnt):
    mesh = plsc.VectorSubcoreMesh(core_axis_name="c", subcore_axis_name="s")
    f = pl.kernel(
        _cnt_body,
        out_type=[jax.ShapeDtypeStruct((_NP, _CW), jnp.float32),
                  jax.ShapeDtypeStruct((_NP, _CW), jnp.float32)],
        mesh=mesh,
        compiler_params=pltpu.CompilerParams(use_tc_tiling_on_sc=False),
        scratch_types=[
            pltpu.VMEM((_GEDGE,), jnp.int32),
            pltpu.VMEM((_GEDGE,), jnp.int32),
            pltpu.VMEM((_GEDGE,), jnp.int32),
            pltpu.VMEM((_GEDGE,), jnp.int32),
            pltpu.VMEM((_GEDGE, _CW), jnp.float32),
            pltpu.VMEM((_GEDGE, _CW), jnp.float32),
            pltpu.VMEM_SHARED((_SPROWS, _CW), jnp.float32),
            pltpu.SemaphoreType.DMA,
            pltpu.SemaphoreType.DMA,
            pltpu.SemaphoreType.DMA,
            pltpu.SemaphoreType.DMA,
        ],
    )
    return f(zt, srcp2, ldst2, zeros_cnt)


def _pool_body(x_hbm, n2s_hbm, s2g_hbm, zeros_hbm, out_hbm,
               s2g_v, nidx_v, gidx_v, rows_v, acc_sh, sem):
    c = lax.axis_index("c")
    t = lax.axis_index("s")

    @pl.when(t == 0)
    def _():
        pltpu.sync_copy(zeros_hbm, acc_sh)

    pltpu.sync_copy(s2g_hbm, s2g_v)
    plsc.subcore_barrier()
    w = c * 16 + t
    base0 = w * _PPT

    def body(i, carry):
        base = base0 + i * _PBLK
        pltpu.sync_copy(x_hbm.at[pl.ds(base, _PBLK)], rows_v)
        pltpu.sync_copy(n2s_hbm.at[pl.ds(base, _PBLK)], nidx_v)
        for j in range(_PBLK // 16):
            sg = nidx_v[pl.ds(j * 16, 16)]
            gidx_v[pl.ds(j * 16, 16)] = plsc.load_gather(s2g_v, [sg])
        pltpu.sync_copy(rows_v, acc_sh.at[gidx_v], add=True)
        return carry

    lax.fori_loop(0, _PPT // _PBLK, body, 0)
    plsc.subcore_barrier()

    @pl.when(t == 0)
    def _():
        pltpu.sync_copy(acc_sh, out_hbm.at[pl.ds(c * 64, 64)])


def _pool_call(x5, n2s_p, s2g_p, zeros_pool):
    mesh = plsc.VectorSubcoreMesh(core_axis_name="c", subcore_axis_name="s")
    f = pl.kernel(
        _pool_body,
        out_type=jax.ShapeDtypeStruct((128, _D), jnp.float32),
        mesh=mesh,
        compiler_params=pltpu.CompilerParams(use_tc_tiling_on_sc=False,
                                             needs_layout_passes=False),
        scratch_types=[
            pltpu.VMEM((512,), jnp.int32),
            pltpu.VMEM((_PBLK,), jnp.int32),
            pltpu.VMEM((_PBLK,), jnp.int32),
            pltpu.VMEM((_PBLK, _D), jnp.float32),
            pltpu.VMEM_SHARED((64, _D), jnp.float32),
            pltpu.SemaphoreType.DMA,
        ],
    )
    return f(x5, n2s_p, s2g_p, zeros_pool)


def kernel(z, edge_index, node_to_subgraph, subgraph_to_graph, node_emb, Wm,
           W_ih, W_hh, b_ih, b_hh, fc1_w, fc1_b, fc2_w, fc2_b, fc3_w, fc3_b):
    L = Wm.shape[0]
    npad = _NP - _N
    z_p = jnp.concatenate(
        [z.astype(jnp.int32), jnp.zeros((npad,), jnp.int32)]).reshape(_NP, 1)
    n2s_p = jnp.concatenate(
        [node_to_subgraph.astype(jnp.int32), jnp.full((npad,), _S, jnp.int32)])
    s2g_p = jnp.concatenate(
        [subgraph_to_graph.astype(jnp.int32), jnp.full((12,), 63, jnp.int32)])
    zeros_agg = jnp.zeros((_ZROWS, _H), jnp.float32)
    zeros_cnt = jnp.zeros((_ZROWS, _CW), jnp.float32)
    zeros_pool = jnp.zeros((64, _D), jnp.float32)
    zt = jnp.concatenate(
        [jnp.ones((_NP, 1), jnp.float32), z_p.astype(jnp.float32),
         jnp.zeros((_NP, _CW - 2), jnp.float32)], axis=1)

    epad = _EPAD - _E
    srcp2 = jnp.concatenate(
        [edge_index[0].astype(jnp.int32), jnp.zeros((epad,), jnp.int32)])
    ldst2 = jnp.concatenate(
        [edge_index[1].astype(jnp.int32), jnp.full((epad,), _TRASH, jnp.int32)])

    acca, accb = _cnt_call(zt, srcp2, ldst2, zeros_cnt)
    ws0 = _split_gru_weights(W_ih[0], W_hh[0], b_ih[0], b_hh[0])
    x, ma, mb = _gru0_call(acca, accb, z_p, node_emb, Wm[0, 0], Wm[0, 1],
                           ws0, Wm[1, 0], Wm[1, 1])
    for l in range(1, L):
        agga, aggb = _agg_call(ma, mb, srcp2, ldst2, zeros_agg)
        ws = _split_gru_weights(W_ih[l], W_hh[l], b_ih[l], b_hh[l])
        if l + 1 < L:
            x, ma, mb = _gru_msg_call(agga, aggb, x, z_p, ws,
                                      Wm[l + 1, 0], Wm[l + 1, 1])
        else:
            x = _gru_final_call(agga, aggb, x, ws)

    pool2 = _pool_call(x, n2s_p, s2g_p, zeros_pool)
    return _mlp_call(pool2, fc1_w, fc1_b, fc2_w, fc2_b, fc3_w, fc3_b)


# speedup vs baseline: 6.6629x; 1.0283x over previous
import functools

import jax
import jax.numpy as jnp
from jax import lax
from jax.experimental import pallas as pl
from jax.experimental.pallas import tpu as pltpu
import jax.experimental.pallas.tpu_sc as plsc

_N = 50000
_E = 800000
_S = 500
_G = 50
_D = 64
_H = 32

_NP = 51200
_TRASH = _NP
_SPROWS = _NP + 16
_ZROWS = _NP // 16

_CHUNK = 128
_GRP = 3
_GEDGE = _GRP * _CHUNK
_GPT = 132
_EPAD = 16 * _GPT * _GEDGE
_ECHUNKROWS = _EPAD // _CHUNK

_CW = 8
_GPT0 = _EPAD // (32 * _GEDGE)

_PBLK = 64
_PPT = _NP // 32


def _sigmoid(x):
    return 1.0 / (1.0 + jnp.exp(-x))


def _elu(x):
    return jnp.where(x > 0, x, jnp.exp(jnp.minimum(x, 0.0)) - 1.0)


_TBLK = 2048
_TGRID = _NP // _TBLK


def _gru0_body(acca_ref, accb_ref, z_ref, e_ref, wm00, wm01, *refs):
    ws = [r[...] for r in refs[:12]]
    w0, w1 = refs[12], refs[13]
    x_out, ma_out, mb_out = refs[14], refs[15], refs[16]
    acc = acca_ref[...] + accb_ref[...]
    cnt = acc[:, 0:1]
    sz = acc[:, 1:2]
    e = e_ref[...]
    dot = functools.partial(jnp.dot, preferred_element_type=jnp.float32)
    m0v = dot(e, wm00[...])
    m1v = dot(e, wm01[...])
    a = (cnt - sz) * m0v + sz * m1v
    h = jnp.broadcast_to(e, a.shape)
    x = _gru_core(a, h, ws)
    x_out[...] = x
    m0 = dot(x, w0[...])
    m1 = dot(x, w1[...])
    m = jnp.where(z_ref[...] == 1, m1, m0)
    ma_out[...] = m[:, 0:_H]
    mb_out[...] = m[:, _H:_D]


def _gru0_call(acca, accb, z_p, node_emb, wm00, wm01, ws, w0, w1):
    return pl.pallas_call(
        _gru0_body,
        grid=(_TGRID,),
        in_specs=[
            pl.BlockSpec((_TBLK, _CW), _ROW),
            pl.BlockSpec((_TBLK, _CW), _ROW),
            pl.BlockSpec((_TBLK, 1), _ROW),
            pl.BlockSpec((1, _D), _FIX),
            pl.BlockSpec((_D, _D), _FIX),
            pl.BlockSpec((_D, _D), _FIX),
            *_WSPECS,
            pl.BlockSpec((_D, _D), _FIX),
            pl.BlockSpec((_D, _D), _FIX),
        ],
        out_specs=[
            pl.BlockSpec((_TBLK, _D), _ROW),
            pl.BlockSpec((_TBLK, _H), _ROW),
            pl.BlockSpec((_TBLK, _H), _ROW),
        ],
        out_shape=[jax.ShapeDtypeStruct((_NP, _D), jnp.float32),
                   jax.ShapeDtypeStruct((_NP, _H), jnp.float32),
                   jax.ShapeDtypeStruct((_NP, _H), jnp.float32)],
    )(acca, accb, z_p, node_emb, wm00, wm01, *ws, w0, w1)


def _gru_core(a, h, ws):
    wir, wiz, win, whr, whz, whn, bir, biz, bin_, bhr, bhz, bhn = ws
    dot = functools.partial(jnp.dot, preferred_element_type=jnp.float32)
    r = _sigmoid(dot(a, wir) + bir + dot(h, whr) + bhr)
    zg = _sigmoid(dot(a, wiz) + biz + dot(h, whz) + bhz)
    n = jnp.tanh(dot(a, win) + bin_ + r * (dot(h, whn) + bhn))
    return (1.0 - zg) * n + zg * h


def _gru_core2(aa, ab, h, ws):
    (wira, wirb, wiza, wizb, wina, winb,
     whr, whz, whn, bir, biz, bin_, bhr, bhz, bhn) = ws
    dot = functools.partial(jnp.dot, preferred_element_type=jnp.float32)
    r = _sigmoid(dot(aa, wira) + dot(ab, wirb) + bir + dot(h, whr) + bhr)
    zg = _sigmoid(dot(aa, wiza) + dot(ab, wizb) + biz + dot(h, whz) + bhz)
    n = jnp.tanh(dot(aa, wina) + dot(ab, winb) + bin_
                 + r * (dot(h, whn) + bhn))
    return (1.0 - zg) * n + zg * h


def _gru_msg_body(agga_ref, aggb_ref, x_ref, z_ref, *refs):
    ws = [r[...] for r in refs[:15]]
    w0, w1 = refs[15], refs[16]
    x_out, ma_out, mb_out = refs[17], refs[18], refs[19]
    x = _gru_core2(agga_ref[...], aggb_ref[...], x_ref[...], ws)
    x_out[...] = x
    dot = functools.partial(jnp.dot, preferred_element_type=jnp.float32)
    m0 = dot(x, w0[...])
    m1 = dot(x, w1[...])
    m = jnp.where(z_ref[...] == 1, m1, m0)
    ma_out[...] = m[:, 0:_H]
    mb_out[...] = m[:, _H:_D]


def _gru_final_body(agga_ref, aggb_ref, x_ref, *refs):
    ws = [r[...] for r in refs[:15]]
    x_out = refs[15]
    x_out[...] = _gru_core2(agga_ref[...], aggb_ref[...], x_ref[...], ws)


def _split_gru_weights(W_ih_l, W_hh_l, b_ih_l, b_hh_l, split_rows=False):
    out = []
    for k in range(3):
        w = W_ih_l[:, k * _D:(k + 1) * _D]
        if split_rows:
            out += [w[0:_H], w[_H:_D]]
        else:
            out += [w]
    out += [W_hh_l[:, k * _D:(k + 1) * _D] for k in range(3)]
    for b in (b_ih_l, b_hh_l):
        out += [b[0:_D].reshape(1, _D), b[_D:2 * _D].reshape(1, _D),
                b[2 * _D:3 * _D].reshape(1, _D)]
    return out


_ROW = lambda i: (i, 0)
_FIX = lambda i: (0, 0)
_WSPECS = ([pl.BlockSpec((_D, _D), _FIX)] * 6
           + [pl.BlockSpec((1, _D), _FIX)] * 6)
_WSPECS2 = ([pl.BlockSpec((_H, _D), _FIX)] * 6
            + [pl.BlockSpec((_D, _D), _FIX)] * 3
            + [pl.BlockSpec((1, _D), _FIX)] * 6)


def _gru_msg_call(agga, aggb, x, z_p, ws, w0, w1):
    return pl.pallas_call(
        _gru_msg_body,
        grid=(_TGRID,),
        in_specs=[
            pl.BlockSpec((_TBLK, _H), _ROW),
            pl.BlockSpec((_TBLK, _H), _ROW),
            pl.BlockSpec((_TBLK, _D), _ROW),
            pl.BlockSpec((_TBLK, 1), _ROW),
            *_WSPECS2,
            pl.BlockSpec((_D, _D), _FIX),
            pl.BlockSpec((_D, _D), _FIX),
        ],
        out_specs=[
            pl.BlockSpec((_TBLK, _D), _ROW),
            pl.BlockSpec((_TBLK, _H), _ROW),
            pl.BlockSpec((_TBLK, _H), _ROW),
        ],
        out_shape=[jax.ShapeDtypeStruct((_NP, _D), jnp.float32),
                   jax.ShapeDtypeStruct((_NP, _H), jnp.float32),
                   jax.ShapeDtypeStruct((_NP, _H), jnp.float32)],
    )(agga, aggb, x, z_p, *ws, w0, w1)


def _gru_final_call(agga, aggb, x, ws):
    return pl.pallas_call(
        _gru_final_body,
        grid=(_TGRID,),
        in_specs=[
            pl.BlockSpec((_TBLK, _H), _ROW),
            pl.BlockSpec((_TBLK, _H), _ROW),
            pl.BlockSpec((_TBLK, _D), _ROW),
            *_WSPECS2,
        ],
        out_specs=pl.BlockSpec((_TBLK, _D), _ROW),
        out_shape=jax.ShapeDtypeStruct((_NP, _D), jnp.float32),
    )(agga, aggb, x, *ws)


def _mlp_body(pool_ref, w1, b1, w2, b2, w3, b3, out_ref):
    p = pool_ref[...]
    x = p[0:64, :] + p[64:128, :]
    dot = functools.partial(jnp.dot, preferred_element_type=jnp.float32)
    h = _elu(dot(x, w1[...]) + b1[...])
    h = _elu(dot(h, w2[...]) + b2[...])
    y = dot(h, w3[...]) + b3[...]
    out_ref[...] = y[0:_G, :]


def _mlp_call(pool2, fc1_w, fc1_b, fc2_w, fc2_b, fc3_w, fc3_b):
    return pl.pallas_call(
        _mlp_body,
        out_shape=jax.ShapeDtypeStruct((_G, 1), jnp.float32),
    )(pool2, fc1_w, fc1_b.reshape(1, 32), fc2_w, fc2_b.reshape(1, 16),
      fc3_w, fc3_b.reshape(1, 1))


def _agg_body(ma_hbm, mb_hbm, srcp_hbm, ldst_hbm, zeros_hbm,
              outa_hbm, outb_hbm,
              sidx0, sidx1, didx0, didx1, rows0, rows1, acc_sh,
              gsem0, gsem1, ssem0, ssem1):
    c = lax.axis_index("c")
    t = lax.axis_index("s")
    sidx = (sidx0, sidx1)
    didx = (didx0, didx1)
    rows = (rows0, rows1)
    gsem = (gsem0, gsem1)
    ssem = (ssem0, ssem1)

    pltpu.sync_copy(zeros_hbm, acc_sh.at[pl.ds(t * _ZROWS, _ZROWS)])

    @pl.when(t == 0)
    def _():
        pltpu.sync_copy(zeros_hbm.at[pl.ds(0, 16)], acc_sh.at[pl.ds(_NP, 16)])

    plsc.subcore_barrier()

    row0 = t * (_GPT * _GRP)

    def load_idx(g, b):
        e0 = (row0 + g * _GRP) * _CHUNK
        pltpu.sync_copy(srcp_hbm.at[pl.ds(e0, _GEDGE)], sidx[b])
        pltpu.sync_copy(ldst_hbm.at[pl.ds(e0, _GEDGE)], didx[b])

    def fire_gathers(b):
        @pl.when(c == 0)
        def _():
            pltpu.async_copy(ma_hbm.at[sidx[b]], rows[b], gsem[b])

        @pl.when(c == 1)
        def _():
            pltpu.async_copy(mb_hbm.at[sidx[b]], rows[b], gsem[b])

    def fire_scatters(b):
        pltpu.async_copy(rows[b], acc_sh.at[didx[b]], ssem[b], add=True)

    def drain(sem, b):
        pltpu.make_async_copy(ma_hbm.at[pl.ds(0, _GEDGE)], rows[b], sem).wait()

    load_idx(0, 0)
    fire_gathers(0)

    def pair_body(p, carry):
        for b in (0, 1):
            g = 2 * p + b
            drain(gsem[b], b)
            fire_scatters(b)

            @pl.when(g > 0)
            def _():
                drain(ssem[1 - b], 1 - b)

            @pl.when(g + 1 < _GPT)
            def _():
                load_idx(g + 1, 1 - b)
                fire_gathers(1 - b)
        return carry

    lax.fori_loop(0, _GPT // 2, pair_body, 0)
    drain(ssem[(_GPT - 1) % 2], (_GPT - 1) % 2)

    plsc.subcore_barrier()
    sl = pl.ds(t * _ZROWS, _ZROWS)

    @pl.when(c == 0)
    def _():
        pltpu.sync_copy(acc_sh.at[sl], outa_hbm.at[sl])

    @pl.when(c == 1)
    def _():
        pltpu.sync_copy(acc_sh.at[sl], outb_hbm.at[sl])


def _agg_call(ma, mb, srcp2, ldst2, zeros_agg):
    mesh = plsc.VectorSubcoreMesh(core_axis_name="c", subcore_axis_name="s")
    f = pl.kernel(
        _agg_body,
        out_type=[jax.ShapeDtypeStruct((_NP, _H), jnp.float32),
                  jax.ShapeDtypeStruct((_NP, _H), jnp.float32)],
        mesh=mesh,
        compiler_params=pltpu.CompilerParams(use_tc_tiling_on_sc=False),
        scratch_types=[
            pltpu.VMEM((_GEDGE,), jnp.int32),
            pltpu.VMEM((_GEDGE,), jnp.int32),
            pltpu.VMEM((_GEDGE,), jnp.int32),
            pltpu.VMEM((_GEDGE,), jnp.int32),
            pltpu.VMEM((_GEDGE, _H), jnp.float32),
            pltpu.VMEM((_GEDGE, _H), jnp.float32),
            pltpu.VMEM_SHARED((_SPROWS, _H), jnp.float32),
            pltpu.SemaphoreType.DMA,
            pltpu.SemaphoreType.DMA,
            pltpu.SemaphoreType.DMA,
            pltpu.SemaphoreType.DMA,
        ],
    )
    return f(ma, mb, srcp2, ldst2, zeros_agg)


def _cnt_body(zt_hbm, srcp_hbm, ldst_hbm, zeros_hbm, outa_hbm, outb_hbm,
              sidx0, sidx1, didx0, didx1, rows0, rows1, acc_sh,
              gsem0, gsem1, ssem0, ssem1):
    c = lax.axis_index("c")
    t = lax.axis_index("s")
    sidx = (sidx0, sidx1)
    didx = (didx0, didx1)
    rows = (rows0, rows1)
    gsem = (gsem0, gsem1)
    ssem = (ssem0, ssem1)

    pltpu.sync_copy(zeros_hbm, acc_sh.at[pl.ds(t * _ZROWS, _ZROWS)])

    @pl.when(t == 0)
    def _():
        pltpu.sync_copy(zeros_hbm.at[pl.ds(0, 16)], acc_sh.at[pl.ds(_NP, 16)])

    plsc.subcore_barrier()

    base0 = (c * 16 + t) * (_GPT0 * _GEDGE)

    def load_idx(g, b):
        e0 = base0 + g * _GEDGE
        pltpu.sync_copy(srcp_hbm.at[pl.ds(e0, _GEDGE)], sidx[b])
        pltpu.sync_copy(ldst_hbm.at[pl.ds(e0, _GEDGE)], didx[b])

    def fire_gathers(b):
        pltpu.async_copy(zt_hbm.at[sidx[b]], rows[b], gsem[b])

    def fire_scatters(b):
        pltpu.async_copy(rows[b], acc_sh.at[didx[b]], ssem[b], add=True)

    def drain(sem, b):
        pltpu.make_async_copy(zt_hbm.at[pl.ds(0, _GEDGE)], rows[b], sem).wait()

    load_idx(0, 0)
    fire_gathers(0)

    def pair_body(p, carry):
        for b in (0, 1):
            g = 2 * p + b
            drain(gsem[b], b)
            fire_scatters(b)

            @pl.when(g > 0)
            def _():
                drain(ssem[1 - b], 1 - b)

            @pl.when(g + 1 < _GPT0)
            def _():
                load_idx(g + 1, 1 - b)
                fire_gathers(1 - b)
        return carry

    lax.fori_loop(0, _GPT0 // 2, pair_body, 0)
    drain(ssem[(_GPT0 - 1) % 2], (_GPT0 - 1) % 2)

    plsc.subcore_barrier()
    sl = pl.ds(t * _ZROWS, _ZROWS)

    @pl.when(c == 0)
    def _():
        pltpu.sync_copy(acc_sh.at[sl], outa_hbm.at[sl])

    @pl.when(c == 1)
    def _():
        pltpu.sync_copy(acc_sh.at[sl], outb_hbm.at[sl])


def _cnt_call(zt, srcp2, ldst2, zeros_cnt):
    mesh = plsc.VectorSubcoreMesh(core_axis_name="c", subcore_axis_name="s")
    f = pl.kernel(
        _cnt_body,
        out_type=[jax.ShapeDtypeStruct((_NP, _CW), jnp.float32),
                  jax.ShapeDtypeStruct((_NP, _CW), jnp.float32)],
        mesh=mesh,
        compiler_params=pltpu.CompilerParams(use_tc_tiling_on_sc=False),
        scratch_types=[
            pltpu.VMEM((_GEDGE,), jnp.int32),
            pltpu.VMEM((_GEDGE,), jnp.int32),
            pltpu.VMEM((_GEDGE,), jnp.int32),
            pltpu.VMEM((_GEDGE,), jnp.int32),
            pltpu.VMEM((_GEDGE, _CW), jnp.float32),
            pltpu.VMEM((_GEDGE, _CW), jnp.float32),
            pltpu.VMEM_SHARED((_SPROWS, _CW), jnp.float32),
            pltpu.SemaphoreType.DMA,
            pltpu.SemaphoreType.DMA,
            pltpu.SemaphoreType.DMA,
            pltpu.SemaphoreType.DMA,
        ],
    )
    return f(zt, srcp2, ldst2, zeros_cnt)


def _pool_body(x_hbm, n2s_hbm, s2g_hbm, zeros_hbm, out_hbm,
               s2g_v, nidx_v, gidx_v, rows_v, acc_sh, sem):
    c = lax.axis_index("c")
    t = lax.axis_index("s")

    @pl.when(t == 0)
    def _():
        pltpu.sync_copy(zeros_hbm, acc_sh)

    pltpu.sync_copy(s2g_hbm, s2g_v)
    plsc.subcore_barrier()
    w = c * 16 + t
    base0 = w * _PPT

    def body(i, carry):
        base = base0 + i * _PBLK
        pltpu.sync_copy(x_hbm.at[pl.ds(base, _PBLK)], rows_v)
        pltpu.sync_copy(n2s_hbm.at[pl.ds(base, _PBLK)], nidx_v)
        for j in range(_PBLK // 16):
            sg = nidx_v[pl.ds(j * 16, 16)]
            gidx_v[pl.ds(j * 16, 16)] = plsc.load_gather(s2g_v, [sg])
        pltpu.sync_copy(rows_v, acc_sh.at[gidx_v], add=True)
        return carry

    lax.fori_loop(0, _PPT // _PBLK, body, 0)
    plsc.subcore_barrier()

    @pl.when(t == 0)
    def _():
        pltpu.sync_copy(acc_sh, out_hbm.at[pl.ds(c * 64, 64)])


def _pool_call(x5, n2s_p, s2g_p, zeros_pool):
    mesh = plsc.VectorSubcoreMesh(core_axis_name="c", subcore_axis_name="s")
    f = pl.kernel(
        _pool_body,
        out_type=jax.ShapeDtypeStruct((128, _D), jnp.float32),
        mesh=mesh,
        compiler_params=pltpu.CompilerParams(use_tc_tiling_on_sc=False,
                                             needs_layout_passes=False),
        scratch_types=[
            pltpu.VMEM((512,), jnp.int32),
            pltpu.VMEM((_PBLK,), jnp.int32),
            pltpu.VMEM((_PBLK,), jnp.int32),
            pltpu.VMEM((_PBLK, _D), jnp.float32),
            pltpu.VMEM_SHARED((64, _D), jnp.float32),
            pltpu.SemaphoreType.DMA,
        ],
    )
    return f(x5, n2s_p, s2g_p, zeros_pool)


def kernel(z, edge_index, node_to_subgraph, subgraph_to_graph, node_emb, Wm,
           W_ih, W_hh, b_ih, b_hh, fc1_w, fc1_b, fc2_w, fc2_b, fc3_w, fc3_b):
    L = Wm.shape[0]
    npad = _NP - _N
    z_p = jnp.concatenate(
        [z.astype(jnp.int32), jnp.zeros((npad,), jnp.int32)]).reshape(_NP, 1)
    n2s_p = jnp.concatenate(
        [node_to_subgraph.astype(jnp.int32), jnp.full((npad,), _S, jnp.int32)])
    s2g_p = jnp.concatenate(
        [subgraph_to_graph.astype(jnp.int32), jnp.full((12,), 63, jnp.int32)])
    zeros_agg = jnp.zeros((_ZROWS, _H), jnp.float32)
    zeros_cnt = jnp.zeros((_ZROWS, _CW), jnp.float32)
    zeros_pool = jnp.zeros((64, _D), jnp.float32)
    zt = jnp.concatenate(
        [jnp.ones((_NP, 1), jnp.float32), z_p.astype(jnp.float32),
         jnp.zeros((_NP, _CW - 2), jnp.float32)], axis=1)

    epad = _EPAD - _E
    srcp2 = jnp.concatenate(
        [edge_index[0].astype(jnp.int32), jnp.zeros((epad,), jnp.int32)])
    ldst2 = jnp.concatenate(
        [edge_index[1].astype(jnp.int32), jnp.full((epad,), _TRASH, jnp.int32)])

    acca, accb = _cnt_call(zt, srcp2, ldst2, zeros_cnt)
    ws0 = _split_gru_weights(W_ih[0], W_hh[0], b_ih[0], b_hh[0])
    x, ma, mb = _gru0_call(acca, accb, z_p, node_emb, Wm[0, 0], Wm[0, 1],
                           ws0, Wm[1, 0], Wm[1, 1])
    for l in range(1, L):
        agga, aggb = _agg_call(ma, mb, srcp2, ldst2, zeros_agg)
        ws = _split_gru_weights(W_ih[l], W_hh[l], b_ih[l], b_hh[l],
                                split_rows=True)
        if l + 1 < L:
            x, ma, mb = _gru_msg_call(agga, aggb, x, z_p, ws,
                                      Wm[l + 1, 0], Wm[l + 1, 1])
        else:
            x = _gru_final_call(agga, aggb, x, ws)

    pool2 = _pool_call(x, n2s_p, s2g_p, zeros_pool)
    return _mlp_call(pool2, fc1_w, fc1_b, fc2_w, fc2_b, fc3_w, fc3_b)
